# Initial kernel scaffold; baseline (speedup 1.0000x reference)
import jax
import jax.numpy as jnp
from jax.experimental import pallas as pl


def kernel(x, W, b):
    raise NotImplementedError("write your pallas kernel here")


# speedup vs baseline: 983.6510x; 983.6510x over previous
import functools

import jax
import jax.numpy as jnp
from jax import lax
from jax.experimental import pallas as pl
from jax.experimental.pallas import tpu as pltpu
from jax.experimental.pallas import tpu_sc as plsc

KNN = 4


def _knn_y_kernel(xt_ref, x_ref, wt_ref, bias_ref,
                  y0, y1, y2, y3, i0, i1, i2, i3):
    a = xt_ref[0]
    xm = x_ref[0]
    tr = a.shape[0]
    n = xm.shape[1]

    dot = lax.dot_general(a, xm, (((1,), (0,)), ((), ())),
                          preferred_element_type=jnp.float32)
    sq_r = jnp.sum(a * a, axis=1, keepdims=True)
    sq_c = jnp.sum(xm * xm, axis=0, keepdims=True)
    d = jnp.sqrt(jnp.maximum(sq_r + sq_c - 2.0 * dot, 0.0))

    iota = lax.broadcasted_iota(jnp.int32, (tr, n), 1)
    gbase = pl.program_id(0) * n
    idx_refs = (i0, i1, i2, i3)
    y_refs = (y0, y1, y2, y3)
    for k in range(KNN):
        m = jnp.min(d, axis=1, keepdims=True)
        ik = jnp.min(jnp.where(d == m, iota, n), axis=1, keepdims=True)
        idx_refs[k][0] = ik + gbase
        d = jnp.where(iota == ik, jnp.inf, d)
        yk = lax.dot_general(a, wt_ref[k], (((1,), (0,)), ((), ())),
                             preferred_element_type=jnp.float32)
        if k == 0:
            yk = yk + bias_ref[...]
        y_refs[k][0] = yk


def _tc_stage(xt, x, wt, bias2, tr):
    B, N, C = xt.shape
    O = wt.shape[2]
    return pl.pallas_call(
        _knn_y_kernel,
        grid=(B, N // tr),
        in_specs=[
            pl.BlockSpec((1, tr, C), lambda b, t: (b, t, 0)),
            pl.BlockSpec((1, C, N), lambda b, t: (b, 0, 0)),
            pl.BlockSpec((KNN, C, O), lambda b, t: (0, 0, 0)),
            pl.BlockSpec((1, O), lambda b, t: (0, 0)),
        ],
        out_specs=(
            [pl.BlockSpec((1, tr, O), lambda b, t: (b, t, 0))] * KNN
            + [pl.BlockSpec((1, tr, 1), lambda b, t: (b, t, 0))] * KNN
        ),
        out_shape=(
            [jax.ShapeDtypeStruct((B, N, O), jnp.float32)] * KNN
            + [jax.ShapeDtypeStruct((B, N, 1), jnp.int32)] * KNN
        ),
    )(xt, x, wt, bias2)


def _make_sc_gather_sum(bn, o, ch):
    info = plsc.get_sparse_core_info()
    nc, ns = info.num_cores, info.num_subcores
    nw = nc * ns
    rows_per_w = bn // nw
    n_chunks = rows_per_w // ch
    mesh = plsc.VectorSubcoreMesh(core_axis_name="c", subcore_axis_name="s")

    @functools.partial(
        pl.kernel,
        out_type=jax.ShapeDtypeStruct((bn, o), jnp.float32),
        mesh=mesh,
        scratch_types=(
            [pltpu.VMEM((ch,), jnp.int32) for _ in range(KNN)]
            + [pltpu.VMEM((ch, o), jnp.float32) for _ in range(KNN)]
            + [pltpu.VMEM((ch, o), jnp.float32)]
            + [pltpu.SemaphoreType.DMA for _ in range(KNN)]
        ),
    )
    def sc_gather_sum(y0, y1, y2, y3, i0, i1, i2, i3, out,
                      ib0, ib1, ib2, ib3, g0, g1, g2, g3, ob,
                      s0, s1, s2, s3):
        wid = lax.axis_index("s") * nc + lax.axis_index("c")
        ys = (y0, y1, y2, y3)
        idx_hbm = (i0, i1, i2, i3)
        ibs = (ib0, ib1, ib2, ib3)
        gbufs = (g0, g1, g2, g3)
        sems = (s0, s1, s2, s3)

        def chunk_body(g, carry):
            base = wid * rows_per_w + g * ch
            for k in range(KNN):
                pltpu.sync_copy(idx_hbm[k].at[pl.ds(base, ch)], ibs[k])
            copies = [pltpu.async_copy(ys[k].at[ibs[k]], gbufs[k], sems[k])
                      for k in range(KNN)]
            for c in copies:
                c.wait()

            def row_body(j, carry2):
                for c16 in range(o // 16):
                    sl = pl.ds(c16 * 16, 16)
                    ob[j, sl] = (gbufs[0][j, sl] + gbufs[1][j, sl]
                                 + gbufs[2][j, sl] + gbufs[3][j, sl])
                return carry2

            lax.fori_loop(0, ch, row_body, 0)
            pltpu.sync_copy(ob, out.at[pl.ds(base, ch)])
            return carry

        lax.fori_loop(0, n_chunks, chunk_body, 0)

    return sc_gather_sum


def kernel(x, W, b):
    B, C, N = x.shape
    O = W.shape[0]
    xt = x.transpose(0, 2, 1)
    wt = W.transpose(2, 1, 0)
    bias2 = b.reshape(1, O)

    outs = _tc_stage(xt, x, wt, bias2, tr=256)
    ys = [o.reshape(B * N, O) for o in outs[:KNN]]
    idxs = [o.reshape(B * N) for o in outs[KNN:]]

    sc = _make_sc_gather_sum(B * N, O, ch=32)
    out_t = sc(*ys, *idxs)
    return out_t.reshape(B, N, O).transpose(0, 2, 1)


# speedup vs baseline: 1056.4003x; 1.0740x over previous
import functools

import jax
import jax.numpy as jnp
from jax import lax
from jax.experimental import pallas as pl
from jax.experimental.pallas import tpu as pltpu
from jax.experimental.pallas import tpu_sc as plsc

KNN = 4


def _knn_y_kernel(xt_ref, x_ref, wt_ref, bias_ref,
                  y0, y1, y2, y3, i0, i1, i2, i3):
    a = xt_ref[0]
    xm = x_ref[0]
    tr = a.shape[0]
    n = xm.shape[1]

    dot = lax.dot_general(a, xm, (((1,), (0,)), ((), ())),
                          preferred_element_type=jnp.float32)
    sq_r = jnp.sum(a * a, axis=1, keepdims=True)
    sq_c = jnp.sum(xm * xm, axis=0, keepdims=True)
    d = jnp.sqrt(jnp.maximum(sq_r + sq_c - 2.0 * dot, 0.0))

    iota = lax.broadcasted_iota(jnp.int32, (tr, n), 1)
    gbase = pl.program_id(0) * n
    idx_refs = (i0, i1, i2, i3)
    y_refs = (y0, y1, y2, y3)
    for k in range(KNN):
        m = jnp.min(d, axis=1, keepdims=True)
        ik = jnp.min(jnp.where(d == m, iota, n), axis=1, keepdims=True)
        idx_refs[k][0] = ik + gbase
        d = jnp.where(iota == ik, jnp.inf, d)
        yk = lax.dot_general(a, wt_ref[k], (((1,), (0,)), ((), ())),
                             preferred_element_type=jnp.float32)
        if k == 0:
            yk = yk + bias_ref[...]
        y_refs[k][0] = yk


def _tc_stage(xt, x, wt, bias2, tr):
    B, N, C = xt.shape
    O = wt.shape[2]
    return pl.pallas_call(
        _knn_y_kernel,
        grid=(B, N // tr),
        in_specs=[
            pl.BlockSpec((1, tr, C), lambda b, t: (b, t, 0)),
            pl.BlockSpec((1, C, N), lambda b, t: (b, 0, 0)),
            pl.BlockSpec((KNN, C, O), lambda b, t: (0, 0, 0)),
            pl.BlockSpec((1, O), lambda b, t: (0, 0)),
        ],
        out_specs=(
            [pl.BlockSpec((1, tr, O), lambda b, t: (b, t, 0))] * KNN
            + [pl.BlockSpec((1, tr, 1), lambda b, t: (b, t, 0))] * KNN
        ),
        out_shape=(
            [jax.ShapeDtypeStruct((B, N, O), jnp.float32)] * KNN
            + [jax.ShapeDtypeStruct((B, N, 1), jnp.int32)] * KNN
        ),
    )(xt, x, wt, bias2)


def _make_sc_gather_sum(bn, o, ch):
    info = plsc.get_sparse_core_info()
    nc, ns = info.num_cores, info.num_subcores
    nw = nc * ns
    rows_per_w = bn // nw
    n_chunks = rows_per_w // ch
    mesh = plsc.VectorSubcoreMesh(core_axis_name="c", subcore_axis_name="s")

    @functools.partial(
        pl.kernel,
        out_type=jax.ShapeDtypeStruct((bn, o), jnp.float32),
        mesh=mesh,
        scratch_types=(
            [pltpu.VMEM((rows_per_w,), jnp.int32) for _ in range(KNN)]
            + [pltpu.VMEM((ch, o), jnp.float32) for _ in range(2 * KNN)]
            + [pltpu.VMEM((ch, o), jnp.float32) for _ in range(2)]
            + [pltpu.SemaphoreType.DMA for _ in range(2)]
            + [pltpu.SemaphoreType.DMA for _ in range(2)]
        ),
    )
    def sc_gather_sum(y0, y1, y2, y3, i0, i1, i2, i3, out,
                      ib0, ib1, ib2, ib3,
                      ga0, ga1, ga2, ga3, gb0, gb1, gb2, gb3,
                      oba, obb, sg0, sg1, so0, so1):
        wid = lax.axis_index("s") * nc + lax.axis_index("c")
        base0 = wid * rows_per_w
        ys = (y0, y1, y2, y3)
        idx_hbm = (i0, i1, i2, i3)
        ibs = (ib0, ib1, ib2, ib3)
        gsets = ((ga0, ga1, ga2, ga3), (gb0, gb1, gb2, gb3))
        obufs = (oba, obb)
        gsems = (sg0, sg1)
        osems = (so0, so1)

        for k in range(KNN):
            pltpu.sync_copy(idx_hbm[k].at[pl.ds(base0, rows_per_w)], ibs[k])

        def fire(g):
            p = g % 2
            return [pltpu.async_copy(
                        ys[k].at[ibs[k].at[pl.ds(g * ch, ch)]],
                        gsets[p][k], gsems[p])
                    for k in range(KNN)]

        gath = {0: fire(0)}
        ocopies = {}
        for g in range(n_chunks):
            p = g % 2
            if g + 1 < n_chunks:
                gath[g + 1] = fire(g + 1)
            for c in gath.pop(g):
                c.wait()
            if g >= 2:
                for c in ocopies.pop(g - 2):
                    c.wait()
            gbufs = gsets[p]
            ob = obufs[p]

            def row_body(j, carry):
                for c16 in range(o // 16):
                    sl = pl.ds(c16 * 16, 16)
                    ob[j, sl] = (gbufs[0][j, sl] + gbufs[1][j, sl]
                                 + gbufs[2][j, sl] + gbufs[3][j, sl])
                return carry

            lax.fori_loop(0, ch, row_body, 0)
            ocopies[g] = [pltpu.async_copy(
                ob, out.at[pl.ds(base0 + g * ch, ch)], osems[p])]
        for g in list(ocopies):
            for c in ocopies.pop(g):
                c.wait()

    return sc_gather_sum


def kernel(x, W, b):
    B, C, N = x.shape
    O = W.shape[0]
    xt = x.transpose(0, 2, 1)
    wt = W.transpose(2, 1, 0)
    bias2 = b.reshape(1, O)

    outs = _tc_stage(xt, x, wt, bias2, tr=256)
    ys = [o.reshape(B * N, O) for o in outs[:KNN]]
    idxs = [o.reshape(B * N) for o in outs[KNN:]]

    sc = _make_sc_gather_sum(B * N, O, ch=32)
    out_t = sc(*ys, *idxs)
    return out_t.reshape(B, N, O).transpose(0, 2, 1)
